# baseline (speedup 1.0000x reference)
import functools

import jax
import jax.numpy as jnp
from jax import lax
from jax.experimental import pallas as pl
from jax.experimental.pallas import tpu as pltpu
from jax.experimental.pallas import tpu_sc as plsc

_PC = pl.pallas_call
_PK = pl.kernel

B = 2
N = 10000
E = 2048
D = 128
H = 16
DH = D // H
EPS = 1e-05

NC = 2
NS = 16
NW = NC * NS
NP = 10240
CW = 16
R = B * N
RB = 2000
EQ = 256


def _ln_body(x_ref, w_ref, b_ref, o_ref):
    x = x_ref[...]
    m = jnp.mean(x, axis=-1, keepdims=True)
    xc = x - m
    v = jnp.mean(xc * xc, axis=-1, keepdims=True)
    o_ref[...] = xc * lax.rsqrt(v + EPS) * w_ref[...] + b_ref[...]


def _layernorm_rows(x2d, w, b):
    return _PC(
        _ln_body,
        grid=(R // RB,),
        in_specs=[
            pl.BlockSpec((RB, D), lambda i: (i, 0)),
            pl.BlockSpec((1, D), lambda i: (0, 0)),
            pl.BlockSpec((1, D), lambda i: (0, 0)),
        ],
        out_specs=pl.BlockSpec((RB, D), lambda i: (i, 0)),
        out_shape=jax.ShapeDtypeStruct((R, D), jnp.float32),
    )(x2d, w.reshape(1, D), b.reshape(1, D))


def _sc_gather(table, gidx):
    G = gidx.shape[0]
    gpw = G // NW
    mesh = plsc.VectorSubcoreMesh(core_axis_name="c", subcore_axis_name="s")

    @functools.partial(
        _PK,
        mesh=mesh,
        out_type=jax.ShapeDtypeStruct((G, D), jnp.float32),
        scratch_types=[
            pltpu.VMEM((gpw,), jnp.int32),
            pltpu.VMEM((gpw, D), jnp.float32),
            pltpu.SemaphoreType.DMA,
        ],
    )
    def gather_k(table_hbm, idx_hbm, out_hbm, idx_v, rows_v, sem):
        wid = lax.axis_index("s") * NC + lax.axis_index("c")
        base = wid * gpw
        pltpu.sync_copy(idx_hbm.at[pl.ds(base, gpw)], idx_v)
        pltpu.async_copy(table_hbm.at[idx_v], rows_v, sem).wait()
        pltpu.sync_copy(rows_v, out_hbm.at[pl.ds(base, gpw)])

    return gather_k(table, gidx)


def _proj_body(kg_ref, qg_ref, ea_ref, w2_ref, b2_ref, Wq_ref, bq_ref,
               Wkv_ref, bkv_ref, We_ref, be_ref, q_o, k_o, v_o):
    ea = ea_ref[...]
    m = jnp.mean(ea, axis=-1, keepdims=True)
    xc = ea - m
    var = jnp.mean(xc * xc, axis=-1, keepdims=True)
    ea_ln = xc * lax.rsqrt(var + EPS) * w2_ref[...] + b2_ref[...]
    e = jnp.dot(ea_ln, We_ref[...], preferred_element_type=jnp.float32) + be_ref[...]
    q = jnp.dot(qg_ref[...], Wq_ref[...], preferred_element_type=jnp.float32) + bq_ref[...] + e
    kv = jnp.dot(kg_ref[...], Wkv_ref[...], preferred_element_type=jnp.float32) + bkv_ref[...]
    q_o[...] = q
    k_o[...] = kv[:, :D]
    v_o[...] = kv[:, D:]


def _projections(kg, qg, ea, ln2_w, ln2_b, Wq, bq, Wkv, bkv, We, be):
    M = B * E
    full = lambda shape: pl.BlockSpec(shape, lambda i: tuple(0 for _ in shape))
    shp = jax.ShapeDtypeStruct((M, D), jnp.float32)
    return _PC(
        _proj_body,
        grid=(1,),
        in_specs=[
            full((M, D)), full((M, D)), full((M, D)),
            full((1, D)), full((1, D)),
            full((D, D)), full((1, D)),
            full((D, 2 * D)), full((1, 2 * D)),
            full((D, D)), full((1, D)),
        ],
        out_specs=[full((M, D)), full((M, D)), full((M, D))],
        out_shape=[shp, shp, shp],
    )(kg, qg, ea, ln2_w.reshape(1, D), ln2_b.reshape(1, D),
      Wq, bq.reshape(1, D), Wkv, bkv.reshape(1, 2 * D), We, be.reshape(1, D))


def _attn_body(q_ref, k_ref, v_ref, vb_ref, w3_ref, b3_ref,
               W1_ref, b1_ref, W2_ref, b2_ref, o_ref):
    q = q_ref[0]
    k = k_ref[0]
    v = v_ref[0]
    head = lax.broadcasted_iota(jnp.int32, (1, D), 1) // DH
    acc = jnp.zeros((EQ, D), jnp.float32)
    scale = 1.0 / (float(DH) ** 0.5)
    for h in range(H):
        msk = (head == h).astype(jnp.float32)
        km = (k * msk).astype(jnp.bfloat16)
        s = lax.dot_general(q.astype(jnp.bfloat16), km, (((1,), (1,)), ((), ())),
                            preferred_element_type=jnp.float32) * scale
        mx = jnp.max(s, axis=-1, keepdims=True)
        p = jnp.exp(s - mx)
        den = jnp.sum(p, axis=-1, keepdims=True)
        oh = lax.dot_general(p.astype(jnp.bfloat16),
                             (v * msk).astype(jnp.bfloat16),
                             (((1,), (0,)), ((), ())),
                             preferred_element_type=jnp.float32)
        acc = acc + oh / den
    hid = vb_ref[0] + acc
    m = jnp.mean(hid, axis=-1, keepdims=True)
    xc = hid - m
    var = jnp.mean(xc * xc, axis=-1, keepdims=True)
    hid = xc * lax.rsqrt(var + EPS) * w3_ref[...] + b3_ref[...]
    f = jnp.maximum(
        jnp.dot(hid, W1_ref[...], preferred_element_type=jnp.float32) + b1_ref[...], 0.0)
    f = jnp.dot(f, W2_ref[...], preferred_element_type=jnp.float32) + b2_ref[...]
    o_ref[0] = f + hid


def _attention(q, k, v, ln3_w, ln3_b, W1, b1, W2, b2):
    wfull = lambda shape: pl.BlockSpec(shape, lambda b_, j: tuple(0 for _ in shape))
    return _PC(
        _attn_body,
        grid=(B, E // EQ),
        in_specs=[
            pl.BlockSpec((1, EQ, D), lambda b_, j: (b_, j, 0)),
            pl.BlockSpec((1, E, D), lambda b_, j: (b_, 0, 0)),
            pl.BlockSpec((1, E, D), lambda b_, j: (b_, 0, 0)),
            pl.BlockSpec((1, EQ, D), lambda b_, j: (b_, j, 0)),
            wfull((1, D)), wfull((1, D)),
            wfull((D, 4 * D)), wfull((1, 4 * D)),
            wfull((4 * D, D)), wfull((1, D)),
        ],
        out_specs=pl.BlockSpec((1, EQ, D), lambda b_, j: (b_, j, 0)),
        out_shape=jax.ShapeDtypeStruct((B, E, D), jnp.float32),
        compiler_params=pltpu.CompilerParams(vmem_limit_bytes=100 * 1024 * 1024),
    )(q, k, v, v, ln3_w.reshape(1, D), ln3_b.reshape(1, D),
      W1, b1.reshape(1, 4 * D), W2, b2.reshape(1, D))


def _sc_scatter(hidden2d, idx1d):
    EPW = E // NS
    RPW = NP // NS
    mesh = plsc.VectorSubcoreMesh(core_axis_name="c", subcore_axis_name="s")
    zs = jnp.zeros((NP, D), jnp.float32)

    @functools.partial(
        _PK,
        mesh=mesh,
        out_type=jax.ShapeDtypeStruct((B * NP, D), jnp.float32),
        scratch_types=[
            pltpu.VMEM((EPW,), jnp.int32),
            pltpu.VMEM((EPW, D), jnp.float32),
            pltpu.VMEM_SHARED((NP, D), jnp.float32),
        ],
    )
    def scatter_k(hid_hbm, idx_hbm, zs_hbm, seg_hbm, idx_v, rows_v, sh_sum):
        c = lax.axis_index("c")
        s_ = lax.axis_index("s")
        ebase = c * E + s_ * EPW
        rbase = s_ * RPW
        obase = c * NP + s_ * RPW
        pltpu.sync_copy(zs_hbm.at[pl.ds(rbase, RPW)], sh_sum.at[pl.ds(rbase, RPW)])
        pltpu.sync_copy(idx_hbm.at[pl.ds(ebase, EPW)], idx_v)
        pltpu.sync_copy(hid_hbm.at[pl.ds(ebase, EPW)], rows_v)
        plsc.subcore_barrier()
        pltpu.sync_copy(rows_v, sh_sum.at[idx_v], add=True)
        plsc.subcore_barrier()
        pltpu.sync_copy(sh_sum.at[pl.ds(rbase, RPW)], seg_hbm.at[pl.ds(obase, RPW)])

    return scatter_k(hidden2d, idx1d, zs)


ECC = 512


def _comb_body(idx_ref, seg_ref, xln_ref, o_ref):
    i = pl.program_id(0)
    b = i // (N // RB)
    nid = (i % (N // RB)) * RB + lax.broadcasted_iota(jnp.int32, (RB, 1), 0)
    cnt = jnp.zeros((RB, 1), jnp.float32)
    for c0 in range(E // ECC):
        ic = idx_ref[pl.ds(b, 1), pl.ds(c0 * ECC, ECC)]
        cnt = cnt + jnp.sum((ic == nid).astype(jnp.float32), axis=1, keepdims=True)
    mean = seg_ref[...] / jnp.maximum(cnt, 1.0)
    o_ref[...] = jnp.where(mean != 0.0, mean, xln_ref[...])


def _combine(idx2, seg2, x_ln):
    return _PC(
        _comb_body,
        grid=(R // RB,),
        in_specs=[
            pl.BlockSpec((B, E), lambda i: (0, 0)),
            pl.BlockSpec((RB, D), lambda i: (i, 0)),
            pl.BlockSpec((RB, D), lambda i: (i, 0)),
        ],
        out_specs=pl.BlockSpec((RB, D), lambda i: (i, 0)),
        out_shape=jax.ShapeDtypeStruct((R, D), jnp.float32),
    )(idx2, seg2, x_ln)


def kernel(x, edge_index, edge_attr, ln1_w, ln1_b, ln2_w, ln2_b, ln3_w, ln3_b,
           Wq, bq, Wkv, bkv, We, be, W1, b1, W2, b2):
    x_ln = _layernorm_rows(x.reshape(R, D), ln1_w, ln1_b)

    ei = edge_index.astype(jnp.int32)
    offs = (jnp.arange(B, dtype=jnp.int32) * N)[:, None]
    src = (ei[:, 0] + offs).reshape(-1)
    dst = (ei[:, 1] + offs).reshape(-1)
    g = _sc_gather(x_ln, jnp.concatenate([src, dst], axis=0))
    kg = g[: B * E]
    qg = g[B * E:]

    q, k, v = _projections(kg, qg, edge_attr.reshape(B * E, D),
                           ln2_w, ln2_b, Wq, bq, Wkv, bkv, We, be)
    hidden = _attention(q.reshape(B, E, D), k.reshape(B, E, D), v.reshape(B, E, D),
                        ln3_w, ln3_b, W1, b1, W2, b2)

    seg = _sc_scatter(hidden.reshape(B * E, D), ei[1].reshape(B * E))
    seg2 = seg.reshape(B, NP, D)[:, :N].reshape(R, D)
    out = _combine(ei[1], seg2, x_ln).reshape(B, N, D)
    return (out, edge_index, edge_attr)


# speedup vs baseline: 1.1332x; 1.1332x over previous
import functools

import jax
import jax.numpy as jnp
from jax import lax
from jax.experimental import pallas as pl
from jax.experimental.pallas import tpu as pltpu
from jax.experimental.pallas import tpu_sc as plsc

_PC = pl.pallas_call
_PK = pl.kernel

B = 2
N = 10000
E = 2048
D = 128
H = 16
DH = D // H
EPS = 1e-05

NC = 2
NS = 16
NW = NC * NS
NP = 10240
CW = 16
R = B * N
RB = 2000
EQ = 256


def _ln_body(x_ref, w_ref, b_ref, o_ref):
    x = x_ref[...]
    m = jnp.mean(x, axis=-1, keepdims=True)
    xc = x - m
    v = jnp.mean(xc * xc, axis=-1, keepdims=True)
    o_ref[...] = xc * lax.rsqrt(v + EPS) * w_ref[...] + b_ref[...]


def _layernorm_rows(x2d, w, b):
    return _PC(
        _ln_body,
        grid=(R // RB,),
        in_specs=[
            pl.BlockSpec((RB, D), lambda i: (i, 0)),
            pl.BlockSpec((1, D), lambda i: (0, 0)),
            pl.BlockSpec((1, D), lambda i: (0, 0)),
        ],
        out_specs=pl.BlockSpec((RB, D), lambda i: (i, 0)),
        out_shape=jax.ShapeDtypeStruct((R, D), jnp.float32),
    )(x2d, w.reshape(1, D), b.reshape(1, D))


def _sc_gather(table, gidx):
    G = gidx.shape[0]
    gpw = G // NW
    mesh = plsc.VectorSubcoreMesh(core_axis_name="c", subcore_axis_name="s")

    @functools.partial(
        _PK,
        mesh=mesh,
        out_type=jax.ShapeDtypeStruct((G, D), jnp.float32),
        scratch_types=[
            pltpu.VMEM((gpw,), jnp.int32),
            pltpu.VMEM((gpw, D), jnp.float32),
            pltpu.SemaphoreType.DMA,
        ],
    )
    def gather_k(table_hbm, idx_hbm, out_hbm, idx_v, rows_v, sem):
        wid = lax.axis_index("s") * NC + lax.axis_index("c")
        base = wid * gpw
        pltpu.sync_copy(idx_hbm.at[pl.ds(base, gpw)], idx_v)
        pltpu.async_copy(table_hbm.at[idx_v], rows_v, sem).wait()
        pltpu.sync_copy(rows_v, out_hbm.at[pl.ds(base, gpw)])

    return gather_k(table, gidx)


def _proj_body(kg_ref, qg_ref, ea_ref, w2_ref, b2_ref, Wq_ref, bq_ref,
               Wkv_ref, bkv_ref, We_ref, be_ref, q_o, k_o, v_o):
    ea = ea_ref[...]
    m = jnp.mean(ea, axis=-1, keepdims=True)
    xc = ea - m
    var = jnp.mean(xc * xc, axis=-1, keepdims=True)
    ea_ln = xc * lax.rsqrt(var + EPS) * w2_ref[...] + b2_ref[...]
    e = jnp.dot(ea_ln, We_ref[...], preferred_element_type=jnp.float32) + be_ref[...]
    q = jnp.dot(qg_ref[...], Wq_ref[...], preferred_element_type=jnp.float32) + bq_ref[...] + e
    kv = jnp.dot(kg_ref[...], Wkv_ref[...], preferred_element_type=jnp.float32) + bkv_ref[...]
    q_o[...] = q
    k_o[...] = kv[:, :D]
    v_o[...] = kv[:, D:]


def _projections(kg, qg, ea, ln2_w, ln2_b, Wq, bq, Wkv, bkv, We, be):
    M = B * E
    full = lambda shape: pl.BlockSpec(shape, lambda i: tuple(0 for _ in shape))
    shp = jax.ShapeDtypeStruct((M, D), jnp.float32)
    return _PC(
        _proj_body,
        grid=(1,),
        in_specs=[
            full((M, D)), full((M, D)), full((M, D)),
            full((1, D)), full((1, D)),
            full((D, D)), full((1, D)),
            full((D, 2 * D)), full((1, 2 * D)),
            full((D, D)), full((1, D)),
        ],
        out_specs=[full((M, D)), full((M, D)), full((M, D))],
        out_shape=[shp, shp, shp],
    )(kg, qg, ea, ln2_w.reshape(1, D), ln2_b.reshape(1, D),
      Wq, bq.reshape(1, D), Wkv, bkv.reshape(1, 2 * D), We, be.reshape(1, D))


def _attn_body(q_ref, k_ref, v_ref, vb_ref, w3_ref, b3_ref,
               W1_ref, b1_ref, W2_ref, b2_ref, o_ref):
    q = q_ref[0]
    k = k_ref[0]
    v = v_ref[0]
    head = lax.broadcasted_iota(jnp.int32, (1, D), 1) // DH
    acc = jnp.zeros((EQ, D), jnp.float32)
    scale = 1.0 / (float(DH) ** 0.5)
    for h in range(H):
        msk = (head == h).astype(jnp.float32)
        km = k * msk
        s = lax.dot_general(q, km, (((1,), (1,)), ((), ())),
                            preferred_element_type=jnp.float32) * scale
        mx = jnp.max(s, axis=-1, keepdims=True)
        p = jnp.exp(s - mx)
        den = jnp.sum(p, axis=-1, keepdims=True)
        oh = lax.dot_general(p, v * msk, (((1,), (0,)), ((), ())),
                             preferred_element_type=jnp.float32)
        acc = acc + oh / den
    hid = vb_ref[0] + acc
    m = jnp.mean(hid, axis=-1, keepdims=True)
    xc = hid - m
    var = jnp.mean(xc * xc, axis=-1, keepdims=True)
    hid = xc * lax.rsqrt(var + EPS) * w3_ref[...] + b3_ref[...]
    f = jnp.maximum(
        jnp.dot(hid, W1_ref[...], preferred_element_type=jnp.float32) + b1_ref[...], 0.0)
    f = jnp.dot(f, W2_ref[...], preferred_element_type=jnp.float32) + b2_ref[...]
    o_ref[0] = f + hid


def _attention(q, k, v, ln3_w, ln3_b, W1, b1, W2, b2):
    wfull = lambda shape: pl.BlockSpec(shape, lambda b_, j: tuple(0 for _ in shape))
    return _PC(
        _attn_body,
        grid=(B, E // EQ),
        in_specs=[
            pl.BlockSpec((1, EQ, D), lambda b_, j: (b_, j, 0)),
            pl.BlockSpec((1, E, D), lambda b_, j: (b_, 0, 0)),
            pl.BlockSpec((1, E, D), lambda b_, j: (b_, 0, 0)),
            pl.BlockSpec((1, EQ, D), lambda b_, j: (b_, j, 0)),
            wfull((1, D)), wfull((1, D)),
            wfull((D, 4 * D)), wfull((1, 4 * D)),
            wfull((4 * D, D)), wfull((1, D)),
        ],
        out_specs=pl.BlockSpec((1, EQ, D), lambda b_, j: (b_, j, 0)),
        out_shape=jax.ShapeDtypeStruct((B, E, D), jnp.float32),
        compiler_params=pltpu.CompilerParams(vmem_limit_bytes=100 * 1024 * 1024),
    )(q, k, v, v, ln3_w.reshape(1, D), ln3_b.reshape(1, D),
      W1, b1.reshape(1, 4 * D), W2, b2.reshape(1, D))


def _sc_scatter(hidden2d, idx1d):
    EPW = E // NS
    RPW = NP // NS
    mesh = plsc.VectorSubcoreMesh(core_axis_name="c", subcore_axis_name="s")
    zs = jnp.zeros((NP, D), jnp.float32)

    @functools.partial(
        _PK,
        mesh=mesh,
        out_type=jax.ShapeDtypeStruct((B * NP, D), jnp.float32),
        scratch_types=[
            pltpu.VMEM((EPW,), jnp.int32),
            pltpu.VMEM((EPW, D), jnp.float32),
            pltpu.VMEM_SHARED((NP, D), jnp.float32),
        ],
    )
    def scatter_k(hid_hbm, idx_hbm, zs_hbm, seg_hbm, idx_v, rows_v, sh_sum):
        c = lax.axis_index("c")
        s_ = lax.axis_index("s")
        ebase = c * E + s_ * EPW
        rbase = s_ * RPW
        obase = c * NP + s_ * RPW
        pltpu.sync_copy(zs_hbm.at[pl.ds(rbase, RPW)], sh_sum.at[pl.ds(rbase, RPW)])
        pltpu.sync_copy(idx_hbm.at[pl.ds(ebase, EPW)], idx_v)
        pltpu.sync_copy(hid_hbm.at[pl.ds(ebase, EPW)], rows_v)
        plsc.subcore_barrier()
        pltpu.sync_copy(rows_v, sh_sum.at[idx_v], add=True)
        plsc.subcore_barrier()
        pltpu.sync_copy(sh_sum.at[pl.ds(rbase, RPW)], seg_hbm.at[pl.ds(obase, RPW)])

    return scatter_k(hidden2d, idx1d, zs)


ECC = 512


def _comb_body(idx_ref, seg_ref, xln_ref, o_ref):
    i = pl.program_id(0)
    b = i // (N // RB)
    nid = (i % (N // RB)) * RB + lax.broadcasted_iota(jnp.int32, (RB, 1), 0)
    cnt = jnp.zeros((RB, 1), jnp.float32)
    for c0 in range(E // ECC):
        ic = idx_ref[pl.ds(b, 1), pl.ds(c0 * ECC, ECC)]
        cnt = cnt + jnp.sum((ic == nid).astype(jnp.float32), axis=1, keepdims=True)
    mean = seg_ref[...] / jnp.maximum(cnt, 1.0)
    o_ref[...] = jnp.where(mean != 0.0, mean, xln_ref[...])


def _combine(idx2, seg2, x_ln):
    return _PC(
        _comb_body,
        grid=(R // RB,),
        in_specs=[
            pl.BlockSpec((B, E), lambda i: (0, 0)),
            pl.BlockSpec((RB, D), lambda i: (i, 0)),
            pl.BlockSpec((RB, D), lambda i: (i, 0)),
        ],
        out_specs=pl.BlockSpec((RB, D), lambda i: (i, 0)),
        out_shape=jax.ShapeDtypeStruct((R, D), jnp.float32),
    )(idx2, seg2, x_ln)


def kernel(x, edge_index, edge_attr, ln1_w, ln1_b, ln2_w, ln2_b, ln3_w, ln3_b,
           Wq, bq, Wkv, bkv, We, be, W1, b1, W2, b2):
    x_ln = _layernorm_rows(x.reshape(R, D), ln1_w, ln1_b)

    ei = edge_index.astype(jnp.int32)
    offs = (jnp.arange(B, dtype=jnp.int32) * N)[:, None]
    src = (ei[:, 0] + offs).reshape(-1)
    dst = (ei[:, 1] + offs).reshape(-1)
    g = _sc_gather(x_ln, jnp.concatenate([src, dst], axis=0))
    kg = g[: B * E]
    qg = g[B * E:]

    q, k, v = _projections(kg, qg, edge_attr.reshape(B * E, D),
                           ln2_w, ln2_b, Wq, bq, Wkv, bkv, We, be)
    hidden = _attention(q.reshape(B, E, D), k.reshape(B, E, D), v.reshape(B, E, D),
                        ln3_w, ln3_b, W1, b1, W2, b2)

    seg = _sc_scatter(hidden.reshape(B * E, D), ei[1].reshape(B * E))
    seg2 = seg.reshape(B, NP, D)[:, :N].reshape(R, D)
    out = _combine(ei[1], seg2, x_ln).reshape(B, N, D)
    return (out, edge_index, edge_attr)
